# baseline (speedup 1.0000x reference)
import functools

import jax
import jax.numpy as jnp
from jax.experimental import pallas as pl
from jax.experimental.pallas import tpu as pltpu
from jax.experimental.pallas import tpu_sc as plsc

D = 384
K = 8192
BR = 256
BC = 2048

_COMMIT = (0.05, 0.25, 0.6)


def _argmin_kernel(z_ref, cb_ref, zn_ref, cn_ref, idx_ref):
    z = z_ref[...]
    cb = cb_ref[...]
    dots = jax.lax.dot_general(z, cb, (((1,), (1,)), ((), ())),
                               preferred_element_type=jnp.float32)
    d = (zn_ref[...] - 2.0 * dots) + cn_ref[...]
    mv = jnp.min(d, axis=1)
    cols = jax.lax.broadcasted_iota(jnp.int32, (BR, K), 1)
    cand = jnp.where(d == mv[:, None], cols, jnp.int32(2147483647))
    idx_ref[0, 0, :] = jnp.min(cand, axis=1)


def _vq_argmin(zf, cb, zn, cn):
    n = zf.shape[0]
    nr = n // BR
    idx3 = pl.pallas_call(
        _argmin_kernel,
        grid=(nr,),
        in_specs=[
            pl.BlockSpec((BR, D), lambda i: (i, 0)),
            pl.BlockSpec((K, D), lambda i: (0, 0)),
            pl.BlockSpec((BR, 1), lambda i: (i, 0)),
            pl.BlockSpec((1, K), lambda i: (0, 0)),
        ],
        out_specs=pl.BlockSpec((1, 1, BR), lambda i: (i, 0, 0)),
        out_shape=jax.ShapeDtypeStruct((nr, 1, BR), jnp.int32),
    )(zf, cb, zn, cn)
    return idx3.reshape(n)


def _loss_kernel(nr, scale, z_ref, q_ref, loss_ref, loss_scr):
    i = pl.program_id(0)
    r = z_ref[...] - q_ref[...]
    s = jnp.sum(r * r)

    @pl.when(i == 0)
    def _():
        loss_scr[0, 0] = s

    @pl.when(i > 0)
    def _():
        loss_scr[0, 0] = loss_scr[0, 0] + s

    @pl.when(i == nr - 1)
    def _():
        loss_ref[...] = jnp.full((1, 1), loss_scr[0, 0] * scale,
                                 jnp.float32)


def _loss(zf, q, commit):
    n = zf.shape[0]
    nr = n // BR
    scale = commit / (n * D)
    out = pl.pallas_call(
        functools.partial(_loss_kernel, nr, scale),
        grid=(nr,),
        in_specs=[
            pl.BlockSpec((BR, D), lambda i: (i, 0)),
            pl.BlockSpec((BR, D), lambda i: (i, 0)),
        ],
        out_specs=pl.BlockSpec((1, 1), lambda i: (0, 0)),
        out_shape=jax.ShapeDtypeStruct((1, 1), jnp.float32),
        scratch_shapes=[pltpu.SMEM((1, 1), jnp.float32)],
    )(zf, q)
    return out[0, 0]


def _sc_gather3(cbs, idxs):
    info = plsc.get_sparse_core_info()
    ncores = info.num_cores
    nw = ncores * info.num_subcores
    ns = [idx.shape[0] for idx in idxs]
    bs = [n // nw for n in ns]
    bmax = max(bs)
    mesh = plsc.VectorSubcoreMesh(core_axis_name="c", subcore_axis_name="s")

    @functools.partial(
        pl.kernel, mesh=mesh,
        out_type=[jax.ShapeDtypeStruct((n, D), jnp.float32) for n in ns],
        scratch_types=[
            pltpu.VMEM((bmax,), jnp.int32),
            pltpu.VMEM((bmax, D), jnp.float32),
            pltpu.SemaphoreType.DMA,
        ],
    )
    def k(cb0_hbm, cb1_hbm, cb2_hbm, i0_hbm, i1_hbm, i2_hbm,
          o0_hbm, o1_hbm, o2_hbm, idx_v, rows_v, sem):
        wid = jax.lax.axis_index("s") * ncores + jax.lax.axis_index("c")
        for cb_hbm, i_hbm, o_hbm, b in (
                (cb0_hbm, i0_hbm, o0_hbm, bs[0]),
                (cb1_hbm, i1_hbm, o1_hbm, bs[1]),
                (cb2_hbm, i2_hbm, o2_hbm, bs[2])):
            base = wid * b
            iv = idx_v.at[pl.ds(0, b)]
            rv = rows_v.at[pl.ds(0, b), :]
            pltpu.sync_copy(i_hbm.at[pl.ds(base, b)], iv)
            pltpu.async_copy(cb_hbm.at[iv], rv, sem).wait()
            pltpu.sync_copy(rv, o_hbm.at[pl.ds(base, b)])

    return k(*cbs, *idxs)


def kernel(l0, l1, l2, cb0, cb1, cb2):
    zs = (l0, l1, l2)
    cbs = (cb0, cb1, cb2)
    zfs, idxs = [], []
    for z, cb in zip(zs, cbs):
        b, t, _ = z.shape
        zf = z.reshape(b * t, D)
        zn = jnp.sum(zf * zf, axis=1, keepdims=True)
        cn = jnp.sum(cb * cb, axis=1)[None, :]
        zfs.append(zf)
        idxs.append(_vq_argmin(zf, cb, zn, cn))
    qs = _sc_gather3(cbs, idxs)
    total_loss = sum(_loss(zf, q, commit)
                     for zf, q, commit in zip(zfs, qs, _COMMIT))
    return (idxs[0].reshape(l0.shape[:2]),
            idxs[1].reshape(l1.shape[:2]),
            idxs[2].reshape(l2.shape[:2]),
            total_loss,
            qs[0].reshape(l0.shape),
            qs[1].reshape(l1.shape),
            qs[2].reshape(l2.shape))


# speedup vs baseline: 1.2315x; 1.2315x over previous
import functools

import jax
import jax.numpy as jnp
from jax.experimental import pallas as pl
from jax.experimental.pallas import tpu as pltpu
from jax.experimental.pallas import tpu_sc as plsc

D = 384
K = 8192
BR = 512
BC = 2048

_COMMIT = (0.05, 0.25, 0.6)


def _argmin_kernel(z_ref, cb_ref, zn_ref, cn_ref, idx_ref):
    z = z_ref[...]
    cb = cb_ref[...]
    dots = jax.lax.dot_general(z, cb, (((1,), (1,)), ((), ())),
                               preferred_element_type=jnp.float32)
    d = (zn_ref[...] - 2.0 * dots) + cn_ref[...]
    idx_ref[0, 0, :] = jnp.argmin(d, axis=1).astype(jnp.int32)


def _vq_argmin(zf, cb, zn, cn):
    n = zf.shape[0]
    nr = n // BR
    idx3 = pl.pallas_call(
        _argmin_kernel,
        grid=(nr,),
        in_specs=[
            pl.BlockSpec((BR, D), lambda i: (i, 0)),
            pl.BlockSpec((K, D), lambda i: (0, 0)),
            pl.BlockSpec((BR, 1), lambda i: (i, 0)),
            pl.BlockSpec((1, K), lambda i: (0, 0)),
        ],
        out_specs=pl.BlockSpec((1, 1, BR), lambda i: (i, 0, 0)),
        out_shape=jax.ShapeDtypeStruct((nr, 1, BR), jnp.int32),
    )(zf, cb, zn, cn)
    return idx3.reshape(n)


def _loss_kernel(nr, scale, z_ref, q_ref, loss_ref, loss_scr):
    i = pl.program_id(0)
    r = z_ref[...] - q_ref[...]
    s = jnp.sum(r * r)

    @pl.when(i == 0)
    def _():
        loss_scr[0, 0] = s

    @pl.when(i > 0)
    def _():
        loss_scr[0, 0] = loss_scr[0, 0] + s

    @pl.when(i == nr - 1)
    def _():
        loss_ref[...] = jnp.full((1, 1), loss_scr[0, 0] * scale,
                                 jnp.float32)


def _loss(zf, q, commit):
    n = zf.shape[0]
    nr = n // BR
    scale = commit / (n * D)
    out = pl.pallas_call(
        functools.partial(_loss_kernel, nr, scale),
        grid=(nr,),
        in_specs=[
            pl.BlockSpec((BR, D), lambda i: (i, 0)),
            pl.BlockSpec((BR, D), lambda i: (i, 0)),
        ],
        out_specs=pl.BlockSpec((1, 1), lambda i: (0, 0)),
        out_shape=jax.ShapeDtypeStruct((1, 1), jnp.float32),
        scratch_shapes=[pltpu.SMEM((1, 1), jnp.float32)],
    )(zf, q)
    return out[0, 0]


def _sc_gather(cb, idx):
    n = idx.shape[0]
    info = plsc.get_sparse_core_info()
    ncores = info.num_cores
    nw = ncores * info.num_subcores
    b_per_w = n // nw
    mesh = plsc.VectorSubcoreMesh(core_axis_name="c", subcore_axis_name="s")

    @functools.partial(
        pl.kernel, mesh=mesh,
        out_type=jax.ShapeDtypeStruct((n, D), cb.dtype),
        scratch_types=[
            pltpu.VMEM((b_per_w,), jnp.int32),
            pltpu.VMEM((b_per_w, D), jnp.float32),
            pltpu.SemaphoreType.DMA,
        ],
    )
    def k(cb_hbm, idx_hbm, out_hbm, idx_v, rows_v, sem):
        wid = jax.lax.axis_index("s") * ncores + jax.lax.axis_index("c")
        base = wid * b_per_w
        pltpu.sync_copy(idx_hbm.at[pl.ds(base, b_per_w)], idx_v)
        pltpu.async_copy(cb_hbm.at[idx_v], rows_v, sem).wait()
        pltpu.sync_copy(rows_v, out_hbm.at[pl.ds(base, b_per_w)])

    return k(cb, idx)


def kernel(l0, l1, l2, cb0, cb1, cb2):
    zs = (l0, l1, l2)
    cbs = (cb0, cb1, cb2)
    zfs, idxs = [], []
    for z, cb in zip(zs, cbs):
        b, t, _ = z.shape
        zf = z.reshape(b * t, D)
        zn = jnp.sum(zf * zf, axis=1, keepdims=True)
        cn = jnp.sum(cb * cb, axis=1)[None, :]
        zfs.append(zf)
        idxs.append(_vq_argmin(zf, cb, zn, cn))
    qs = [_sc_gather(cb, idx) for cb, idx in zip(cbs, idxs)]
    total_loss = sum(_loss(zf, q, commit)
                     for zf, q, commit in zip(zfs, qs, _COMMIT))
    return (idxs[0].reshape(l0.shape[:2]),
            idxs[1].reshape(l1.shape[:2]),
            idxs[2].reshape(l2.shape[:2]),
            total_loss,
            qs[0].reshape(l0.shape),
            qs[1].reshape(l1.shape),
            qs[2].reshape(l2.shape))


# speedup vs baseline: 1.3447x; 1.0919x over previous
import functools

import jax
import jax.numpy as jnp
from jax.experimental import pallas as pl
from jax.experimental.pallas import tpu as pltpu
from jax.experimental.pallas import tpu_sc as plsc

D = 384
K = 8192
BR = 512
BC = 2048

_COMMIT = (0.05, 0.25, 0.6)


def _argmin_kernel(z_ref, cb_ref, zn_ref, cn_ref, idx_ref):
    z = z_ref[...]
    cb = cb_ref[...]
    dots2 = jax.lax.dot_general(z * -2.0, cb, (((1,), (1,)), ((), ())),
                                preferred_element_type=jnp.float32)
    d = (zn_ref[...] + dots2) + cn_ref[...]
    idx_ref[0, 0, :] = jnp.argmin(d, axis=1).astype(jnp.int32)


def _vq_argmin(zf, cb, zn, cn):
    n = zf.shape[0]
    nr = n // BR
    idx3 = pl.pallas_call(
        _argmin_kernel,
        grid=(nr,),
        in_specs=[
            pl.BlockSpec((BR, D), lambda i: (i, 0)),
            pl.BlockSpec((K, D), lambda i: (0, 0)),
            pl.BlockSpec((BR, 1), lambda i: (i, 0)),
            pl.BlockSpec((1, K), lambda i: (0, 0)),
        ],
        out_specs=pl.BlockSpec((1, 1, BR), lambda i: (i, 0, 0)),
        out_shape=jax.ShapeDtypeStruct((nr, 1, BR), jnp.int32),
    )(zf, cb, zn, cn)
    return idx3.reshape(n)


def _loss_kernel(nr, scale, z_ref, q_ref, loss_ref, loss_scr):
    i = pl.program_id(0)
    r = z_ref[...] - q_ref[...]
    s = jnp.sum(r * r)

    @pl.when(i == 0)
    def _():
        loss_scr[0, 0] = s

    @pl.when(i > 0)
    def _():
        loss_scr[0, 0] = loss_scr[0, 0] + s

    @pl.when(i == nr - 1)
    def _():
        loss_ref[...] = jnp.full((1, 1), loss_scr[0, 0] * scale,
                                 jnp.float32)


def _loss(zf, q, commit):
    n = zf.shape[0]
    nr = n // BR
    scale = commit / (n * D)
    out = pl.pallas_call(
        functools.partial(_loss_kernel, nr, scale),
        grid=(nr,),
        in_specs=[
            pl.BlockSpec((BR, D), lambda i: (i, 0)),
            pl.BlockSpec((BR, D), lambda i: (i, 0)),
        ],
        out_specs=pl.BlockSpec((1, 1), lambda i: (0, 0)),
        out_shape=jax.ShapeDtypeStruct((1, 1), jnp.float32),
        scratch_shapes=[pltpu.SMEM((1, 1), jnp.float32)],
    )(zf, q)
    return out[0, 0]


def _sc_gather(cb, idx):
    n = idx.shape[0]
    info = plsc.get_sparse_core_info()
    ncores = info.num_cores
    nw = ncores * info.num_subcores
    b_per_w = n // nw
    mesh = plsc.VectorSubcoreMesh(core_axis_name="c", subcore_axis_name="s")

    @functools.partial(
        pl.kernel, mesh=mesh,
        out_type=jax.ShapeDtypeStruct((n, D), cb.dtype),
        scratch_types=[
            pltpu.VMEM((b_per_w,), jnp.int32),
            pltpu.VMEM((b_per_w, D), jnp.float32),
            pltpu.SemaphoreType.DMA,
        ],
    )
    def k(cb_hbm, idx_hbm, out_hbm, idx_v, rows_v, sem):
        wid = jax.lax.axis_index("s") * ncores + jax.lax.axis_index("c")
        base = wid * b_per_w
        pltpu.sync_copy(idx_hbm.at[pl.ds(base, b_per_w)], idx_v)
        pltpu.async_copy(cb_hbm.at[idx_v], rows_v, sem).wait()
        pltpu.sync_copy(rows_v, out_hbm.at[pl.ds(base, b_per_w)])

    return k(cb, idx)


def kernel(l0, l1, l2, cb0, cb1, cb2):
    zs = (l0, l1, l2)
    cbs = (cb0, cb1, cb2)
    zfs, idxs = [], []
    for z, cb in zip(zs, cbs):
        b, t, _ = z.shape
        zf = z.reshape(b * t, D)
        zn = jnp.sum(zf * zf, axis=1, keepdims=True)
        cn = jnp.sum(cb * cb, axis=1)[None, :]
        zfs.append(zf)
        idxs.append(_vq_argmin(zf, cb, zn, cn))
    qs = [_sc_gather(cb, idx) for cb, idx in zip(cbs, idxs)]
    total_loss = sum(_loss(zf, q, commit)
                     for zf, q, commit in zip(zfs, qs, _COMMIT))
    return (idxs[0].reshape(l0.shape[:2]),
            idxs[1].reshape(l1.shape[:2]),
            idxs[2].reshape(l2.shape[:2]),
            total_loss,
            qs[0].reshape(l0.shape),
            qs[1].reshape(l1.shape),
            qs[2].reshape(l2.shape))
